# baseline (speedup 1.0000x reference)
import functools

import jax
import jax.numpy as jnp
from jax import lax
from jax.experimental import pallas as pl
from jax.experimental.pallas import tpu as pltpu
from jax.experimental.pallas import tpu_sc as plsc

N = 10000
E = 320000
D = 128
G = 16
L = 8

NC = 2
NS = 16
NW = NC * NS
CH = 128
NG = CH // 16
NPAD = 10240
LR = NPAD // NW
DRR = NPAD // NW // 8
SLACK = 128
EPP = E + SLACK
NOFF = 48
RB = 1000
NRB = N // RB

_SELU_SCALE = 1.0507009873554805
_SELU_ALPHA = 1.6732632423543772


def _qkvs_first_body(x_ref, w_ref, b_ref, q_ref, k_ref, v_ref, s_ref):
    h = x_ref[...]
    y = jnp.dot(h, w_ref[...], preferred_element_type=jnp.float32) + b_ref[...]
    q_ref[...] = y[:, 0:D]
    k_ref[...] = y[:, D:2 * D]
    v_ref[...] = y[:, 2 * D:3 * D]
    s_ref[...] = y[:, 3 * D:4 * D]


def _qkvs_mid_body(num_ref, den_ref, skip_ref, w_ref, b_ref,
                   q_ref, k_ref, v_ref, s_ref):
    num = num_ref[...]
    den = den_ref[:, 0:1] + 1e-16
    h = num / den + skip_ref[...]
    h = _SELU_SCALE * jnp.where(h > 0, h, _SELU_ALPHA * (jnp.exp(h) - 1.0))
    y = jnp.dot(h, w_ref[...], preferred_element_type=jnp.float32) + b_ref[...]
    q_ref[...] = y[:, 0:D]
    k_ref[...] = y[:, D:2 * D]
    v_ref[...] = y[:, 2 * D:3 * D]
    s_ref[...] = y[:, 3 * D:4 * D]


def _run_qkvs_first(x, w, b):
    return pl.pallas_call(
        _qkvs_first_body,
        grid=(NRB,),
        in_specs=[
            pl.BlockSpec((RB, D), lambda i: (i, 0)),
            pl.BlockSpec((D, 4 * D), lambda i: (0, 0)),
            pl.BlockSpec((1, 4 * D), lambda i: (0, 0)),
        ],
        out_specs=[pl.BlockSpec((RB, D), lambda i: (i, 0))] * 4,
        out_shape=[jax.ShapeDtypeStruct((N, D), jnp.float32)] * 4,
    )(x, w, b)


def _run_qkvs_mid(num, den, skip, w, b):
    return pl.pallas_call(
        _qkvs_mid_body,
        grid=(NRB,),
        in_specs=[
            pl.BlockSpec((RB, D), lambda i: (i, 0)),
            pl.BlockSpec((RB, 16), lambda i: (i, 0)),
            pl.BlockSpec((RB, D), lambda i: (i, 0)),
            pl.BlockSpec((D, 4 * D), lambda i: (0, 0)),
            pl.BlockSpec((1, 4 * D), lambda i: (0, 0)),
        ],
        out_specs=[pl.BlockSpec((RB, D), lambda i: (i, 0))] * 4,
        out_shape=[jax.ShapeDtypeStruct((N, D), jnp.float32)] * 4,
    )(num, den, skip, w, b)


def _final_body(num_ref, den_ref, skip_ref, bb_ref, lw_ref, lb_ref,
                pooled_ref, out_ref):
    i = pl.program_id(0)
    num = num_ref[...]
    den = den_ref[:, 0:1] + 1e-16
    h = num / den + skip_ref[...]

    @pl.when(i == 0)
    def _():
        pooled_ref[...] = jnp.full((G, D), -jnp.inf, jnp.float32)

    bb = bb_ref[...]
    for g in range(G):
        vals = jnp.where(bb == g, h, -jnp.inf)
        mg = jnp.max(vals, axis=0, keepdims=True)
        pooled_ref[pl.ds(g, 1), :] = jnp.maximum(pooled_ref[pl.ds(g, 1), :], mg)

    @pl.when(i == NRB - 1)
    def _():
        p = pooled_ref[...]
        yv = jnp.dot(p, lw_ref[...], preferred_element_type=jnp.float32)
        yv = yv + lb_ref[...]
        out_ref[...] = 1.0 / (1.0 + jnp.exp(-yv))


def _run_final(num, den, skip, bb, lw, lb):
    _, out = pl.pallas_call(
        _final_body,
        grid=(NRB,),
        in_specs=[
            pl.BlockSpec((RB, D), lambda i: (i, 0)),
            pl.BlockSpec((RB, 16), lambda i: (i, 0)),
            pl.BlockSpec((RB, D), lambda i: (i, 0)),
            pl.BlockSpec((RB, D), lambda i: (i, 0)),
            pl.BlockSpec((D, 1), lambda i: (0, 0)),
            pl.BlockSpec((1, 1), lambda i: (0, 0)),
        ],
        out_specs=[
            pl.BlockSpec((G, D), lambda i: (0, 0)),
            pl.BlockSpec((G, 1), lambda i: (0, 0)),
        ],
        out_shape=[
            jax.ShapeDtypeStruct((G, D), jnp.float32),
            jax.ShapeDtypeStruct((G, 1), jnp.float32),
        ],
    )(num, den, skip, bb, lw, lb)
    return out


def _sc_edge_body(q_hbm, k_hbm, v_hbm, dst_hbm, src_hbm, off_hbm,
                  outnum_hbm, outden_hbm,
                  dstc, srcc, qloc, kbuf, vbuf, accloc, denloc, offb):
    c = lax.axis_index("c")
    s = lax.axis_index("s")
    g = c * NS + s
    base = g * LR
    iota = lax.iota(jnp.int32, 16)
    z16 = jnp.zeros((16,), jnp.float32)
    lane0 = iota == 0

    pltpu.sync_copy(off_hbm, offb)

    def _scalar_at(pos):
        w = offb[0, pl.ds((pos // 16) * 16, 16)]
        spl = w.at[jnp.broadcast_to(pos % 16, (16,)).astype(jnp.int32)].get(
            mode=lax.GatherScatterMode.PROMISE_IN_BOUNDS)
        return spl[0]

    off0 = _scalar_at(g)
    off1 = _scalar_at(g + 1)
    off0a = lax.bitwise_and(off0, jnp.int32(~127))
    nch = (off1 - off0a + (CH - 1)) // CH

    def zrow(i, carry):
        for t in range(D // 16):
            accloc[i, pl.ds(t * 16, 16)] = z16
        return carry

    lax.fori_loop(0, LR, zrow, 0)

    def zden(i, carry):
        for t in range(D // 16):
            denloc[i, pl.ds(t * 16, 16)] = z16
        return carry

    lax.fori_loop(0, DRR, zden, 0)

    pltpu.sync_copy(q_hbm.at[pl.ds(base, LR)], qloc)

    inv = jnp.float32(1.0 / (D ** 0.5))

    def chunk(j, carry):
        st = pl.multiple_of(off0a + j * CH, CH)
        pltpu.sync_copy(dst_hbm.at[:, pl.ds(st, CH)], dstc)
        pltpu.sync_copy(src_hbm.at[:, pl.ds(st, CH)], srcc)
        idx_s = srcc.at[0]
        pltpu.sync_copy(k_hbm.at[idx_s], kbuf)
        pltpu.sync_copy(v_hbm.at[idx_s], vbuf)
        for gi in range(NG):
            rows = iota + (gi * 16)
            dstv = dstc[0, pl.ds(gi * 16, 16)]
            ldv = jnp.minimum(jnp.maximum(dstv - base, 0), LR - 1)

            def dstep(d, accs):
                a0, a1 = accs
                c0 = jnp.broadcast_to(d * 2, (16,)).astype(jnp.int32)
                c1 = c0 + 1
                a0 = a0 + (plsc.load_gather(qloc, [ldv, c0]) *
                           plsc.load_gather(kbuf, [rows, c0]))
                a1 = a1 + (plsc.load_gather(qloc, [ldv, c1]) *
                           plsc.load_gather(kbuf, [rows, c1]))
                return (a0, a1)

            a0, a1 = lax.fori_loop(0, D // 2, dstep, (z16, z16), unroll=8)
            eidx = st + (gi * 16) + iota
            ok = jnp.logical_and(eidx >= off0, eidx < off1)
            ex16 = jnp.exp((a0 + a1) * inv) * jnp.where(ok, jnp.float32(1.0),
                                                        jnp.float32(0.0))

            def sstep(d, carry2):
                col = jnp.broadcast_to(d, (16,)).astype(jnp.int32)
                vc = plsc.load_gather(vbuf, [rows, col])
                plsc.addupdate_scatter(accloc, [ldv, col], vc * ex16)
                return carry2

            lax.fori_loop(0, D, sstep, 0, unroll=8)
            plsc.addupdate_scatter(
                denloc, [lax.shift_right_logical(ldv, 3),
                         lax.bitwise_and(ldv, 7) * 16], ex16)
        return carry

    lax.fori_loop(0, nch, chunk, 0)
    pltpu.sync_copy(accloc, outnum_hbm.at[pl.ds(base, LR)])
    pltpu.sync_copy(denloc, outden_hbm.at[pl.ds(g * DRR, DRR)])


def _run_sc_edge(q, k, v, dst2, src2, off2):
    mesh = plsc.VectorSubcoreMesh(core_axis_name="c", subcore_axis_name="s",
                                  num_cores=NC, num_subcores=NS)
    kern = pl.kernel(
        _sc_edge_body,
        out_type=[
            jax.ShapeDtypeStruct((NPAD, D), jnp.float32),
            jax.ShapeDtypeStruct((NPAD // 8, D), jnp.float32),
        ],
        mesh=mesh,
        compiler_params=pltpu.CompilerParams(needs_layout_passes=False),
        scratch_types=[
            pltpu.VMEM((1, CH), jnp.int32),
            pltpu.VMEM((1, CH), jnp.int32),
            pltpu.VMEM((LR, D), jnp.float32),
            pltpu.VMEM((CH, D), jnp.float32),
            pltpu.VMEM((CH, D), jnp.float32),
            pltpu.VMEM((LR, D), jnp.float32),
            pltpu.VMEM((DRR, D), jnp.float32),
            pltpu.VMEM((1, NOFF), jnp.int32),
        ],
    )
    return kern(q, k, v, dst2, src2, off2)


def kernel(x, edge_index, batch, Wq, bq, Wk, bk, Wv, bv, Ws, bs, lin_W, lin_b):
    dsts, srcs = lax.sort((edge_index[1], edge_index[0]), num_keys=1)
    dst2 = jnp.concatenate(
        [dsts, jnp.full((SLACK,), N, jnp.int32)]).reshape(1, EPP)
    src2 = jnp.concatenate(
        [srcs, jnp.zeros((SLACK,), jnp.int32)]).reshape(1, EPP)
    off = jnp.searchsorted(dsts, jnp.arange(NW + 1, dtype=jnp.int32) * LR)
    off2 = jnp.pad(off.astype(jnp.int32), (0, NOFF - (NW + 1)),
                   constant_values=E).reshape(1, NOFF)

    bb = jnp.broadcast_to(batch[:, None], (N, D))
    wcat = jnp.concatenate([Wq, Wk, Wv, Ws], axis=2)
    bcat = jnp.concatenate([bq, bk, bv, bs], axis=1)
    bcat = bcat.reshape(L, 1, 4 * D)

    skip = None
    num = den = None
    for l in range(L):
        if l == 0:
            q, k, v, skip = _run_qkvs_first(x, wcat[0], bcat[0])
        else:
            q, k, v, skip = _run_qkvs_mid(num, den, skip, wcat[l], bcat[l])
        qp = jnp.pad(q, ((0, NPAD - N), (0, 0)))
        num, den_raw = _run_sc_edge(qp, k, v, dst2, src2, off2)
        den = den_raw.reshape(NPAD, 16)
    return _run_final(num, den, skip, bb, lin_W, lin_b.reshape(1, 1))


# speedup vs baseline: 2.0171x; 2.0171x over previous
import functools

import jax
import jax.numpy as jnp
from jax import lax
from jax.experimental import pallas as pl
from jax.experimental.pallas import tpu as pltpu
from jax.experimental.pallas import tpu_sc as plsc

N = 10000
E = 320000
D = 128
G = 16
L = 8

NC = 2
NS = 16
NW = NC * NS
CH = 128
NG = CH // 16
NPAD = 10240
LR = NPAD // NW
DRR = NPAD // NW // 8
SLACK = 128
EPP = E + SLACK
NOFF = 48
RB = 1000
NRB = N // RB

_SELU_SCALE = 1.0507009873554805
_SELU_ALPHA = 1.6732632423543772


def _qkvs_first_body(x_ref, w_ref, b_ref, q_ref, k_ref, v_ref, s_ref):
    h = x_ref[...]
    y = jnp.dot(h, w_ref[...], preferred_element_type=jnp.float32) + b_ref[...]
    q_ref[...] = y[:, 0:D]
    k_ref[...] = y[:, D:2 * D]
    v_ref[...] = y[:, 2 * D:3 * D]
    s_ref[...] = y[:, 3 * D:4 * D]


def _qkvs_mid_body(num_ref, den_ref, skip_ref, w_ref, b_ref,
                   q_ref, k_ref, v_ref, s_ref):
    num = num_ref[...]
    den = den_ref[:, 0:1] + 1e-16
    h = num / den + skip_ref[...]
    h = _SELU_SCALE * jnp.where(h > 0, h, _SELU_ALPHA * (jnp.exp(h) - 1.0))
    y = jnp.dot(h, w_ref[...], preferred_element_type=jnp.float32) + b_ref[...]
    q_ref[...] = y[:, 0:D]
    k_ref[...] = y[:, D:2 * D]
    v_ref[...] = y[:, 2 * D:3 * D]
    s_ref[...] = y[:, 3 * D:4 * D]


def _run_qkvs_first(x, w, b):
    return pl.pallas_call(
        _qkvs_first_body,
        grid=(NRB,),
        in_specs=[
            pl.BlockSpec((RB, D), lambda i: (i, 0)),
            pl.BlockSpec((D, 4 * D), lambda i: (0, 0)),
            pl.BlockSpec((1, 4 * D), lambda i: (0, 0)),
        ],
        out_specs=[pl.BlockSpec((RB, D), lambda i: (i, 0))] * 4,
        out_shape=[jax.ShapeDtypeStruct((N, D), jnp.float32)] * 4,
    )(x, w, b)


def _run_qkvs_mid(num, den, skip, w, b):
    return pl.pallas_call(
        _qkvs_mid_body,
        grid=(NRB,),
        in_specs=[
            pl.BlockSpec((RB, D), lambda i: (i, 0)),
            pl.BlockSpec((RB, 16), lambda i: (i, 0)),
            pl.BlockSpec((RB, D), lambda i: (i, 0)),
            pl.BlockSpec((D, 4 * D), lambda i: (0, 0)),
            pl.BlockSpec((1, 4 * D), lambda i: (0, 0)),
        ],
        out_specs=[pl.BlockSpec((RB, D), lambda i: (i, 0))] * 4,
        out_shape=[jax.ShapeDtypeStruct((N, D), jnp.float32)] * 4,
    )(num, den, skip, w, b)


def _final_body(num_ref, den_ref, skip_ref, bb_ref, lw_ref, lb_ref,
                pooled_ref, out_ref):
    i = pl.program_id(0)
    num = num_ref[...]
    den = den_ref[:, 0:1] + 1e-16
    h = num / den + skip_ref[...]

    @pl.when(i == 0)
    def _():
        pooled_ref[...] = jnp.full((G, D), -jnp.inf, jnp.float32)

    bb = bb_ref[...]
    for g in range(G):
        vals = jnp.where(bb == g, h, -jnp.inf)
        mg = jnp.max(vals, axis=0, keepdims=True)
        pooled_ref[pl.ds(g, 1), :] = jnp.maximum(pooled_ref[pl.ds(g, 1), :], mg)

    @pl.when(i == NRB - 1)
    def _():
        p = pooled_ref[...]
        yv = jnp.dot(p, lw_ref[...], preferred_element_type=jnp.float32)
        yv = yv + lb_ref[...]
        out_ref[...] = 1.0 / (1.0 + jnp.exp(-yv))


def _run_final(num, den, skip, bb, lw, lb):
    _, out = pl.pallas_call(
        _final_body,
        grid=(NRB,),
        in_specs=[
            pl.BlockSpec((RB, D), lambda i: (i, 0)),
            pl.BlockSpec((RB, 16), lambda i: (i, 0)),
            pl.BlockSpec((RB, D), lambda i: (i, 0)),
            pl.BlockSpec((RB, D), lambda i: (i, 0)),
            pl.BlockSpec((D, 1), lambda i: (0, 0)),
            pl.BlockSpec((1, 1), lambda i: (0, 0)),
        ],
        out_specs=[
            pl.BlockSpec((G, D), lambda i: (0, 0)),
            pl.BlockSpec((G, 1), lambda i: (0, 0)),
        ],
        out_shape=[
            jax.ShapeDtypeStruct((G, D), jnp.float32),
            jax.ShapeDtypeStruct((G, 1), jnp.float32),
        ],
    )(num, den, skip, bb, lw, lb)
    return out


def _sc_edge_body(q_hbm, k_hbm, v_hbm, dst_hbm, src_hbm, off_hbm,
                  outnum_hbm, outden_hbm,
                  dstc, srcc, qloc, kbuf, vbuf, accloc, denloc, offb):
    c = lax.axis_index("c")
    s = lax.axis_index("s")
    g = c * NS + s
    base = g * LR
    iota = lax.iota(jnp.int32, 16)
    z16 = jnp.zeros((16,), jnp.float32)
    lane0 = iota == 0

    pltpu.sync_copy(off_hbm, offb)

    def _scalar_at(pos):
        w = offb[0, pl.ds((pos // 16) * 16, 16)]
        spl = w.at[jnp.broadcast_to(pos % 16, (16,)).astype(jnp.int32)].get(
            mode=lax.GatherScatterMode.PROMISE_IN_BOUNDS)
        return spl[0]

    off0 = _scalar_at(g)
    off1 = _scalar_at(g + 1)
    off0a = lax.bitwise_and(off0, jnp.int32(~127))
    nch = (off1 - off0a + (CH - 1)) // CH

    def zrow(i, carry):
        for t in range(D // 16):
            accloc[i, pl.ds(t * 16, 16)] = z16
        return carry

    lax.fori_loop(0, LR, zrow, 0)

    def zden(i, carry):
        for t in range(D // 16):
            denloc[i, pl.ds(t * 16, 16)] = z16
        return carry

    lax.fori_loop(0, DRR, zden, 0)

    pltpu.sync_copy(q_hbm.at[pl.ds(base, LR)], qloc)

    inv = jnp.float32(1.0 / (D ** 0.5))

    def chunk(j, carry):
        st = pl.multiple_of(off0a + j * CH, CH)
        pltpu.sync_copy(dst_hbm.at[:, pl.ds(st, CH)], dstc)
        pltpu.sync_copy(src_hbm.at[:, pl.ds(st, CH)], srcc)
        idx_s = srcc.at[0]
        pltpu.sync_copy(k_hbm.at[idx_s], kbuf)
        pltpu.sync_copy(v_hbm.at[idx_s], vbuf)
        for gi in range(NG):
            rows = iota + (gi * 16)
            dstv = dstc[0, pl.ds(gi * 16, 16)]
            ldv = jnp.minimum(jnp.maximum(dstv - base, 0), LR - 1)

            def dstep(d, accs):
                a0, a1 = accs
                c0 = jnp.broadcast_to(d * 2, (16,)).astype(jnp.int32)
                c1 = c0 + 1
                a0 = a0 + (plsc.load_gather(qloc, [ldv, c0]) *
                           plsc.load_gather(kbuf, [rows, c0]))
                a1 = a1 + (plsc.load_gather(qloc, [ldv, c1]) *
                           plsc.load_gather(kbuf, [rows, c1]))
                return (a0, a1)

            a0, a1 = lax.fori_loop(0, D // 2, dstep, (z16, z16), unroll=8)
            eidx = st + (gi * 16) + iota
            ok = jnp.logical_and(eidx >= off0, eidx < off1)
            ex16 = jnp.exp((a0 + a1) * inv) * jnp.where(ok, jnp.float32(1.0),
                                                        jnp.float32(0.0))

            for e in range(16):
                exs = ex16.at[jnp.broadcast_to(jnp.int32(e), (16,))].get(
                    mode=lax.GatherScatterMode.PROMISE_IN_BOUNDS)
                ld = ldv[e]
                r = gi * 16 + e
                for t in range(D // 16):
                    plsc.addupdate(accloc.at[ld, pl.ds(t * 16, 16)],
                                   vbuf[r, pl.ds(t * 16, 16)] * exs)
                ldr = lax.shift_right_logical(ld, 3)
                ldc = pl.multiple_of(lax.bitwise_and(ld, 7) * 16, 16)
                plsc.addupdate(denloc.at[ldr, pl.ds(ldc, 16)],
                               jnp.where(lane0, exs, jnp.float32(0.0)))
        return carry

    lax.fori_loop(0, nch, chunk, 0)
    pltpu.sync_copy(accloc, outnum_hbm.at[pl.ds(base, LR)])
    pltpu.sync_copy(denloc, outden_hbm.at[pl.ds(g * DRR, DRR)])


def _run_sc_edge(q, k, v, dst2, src2, off2):
    mesh = plsc.VectorSubcoreMesh(core_axis_name="c", subcore_axis_name="s",
                                  num_cores=NC, num_subcores=NS)
    kern = pl.kernel(
        _sc_edge_body,
        out_type=[
            jax.ShapeDtypeStruct((NPAD, D), jnp.float32),
            jax.ShapeDtypeStruct((NPAD // 8, D), jnp.float32),
        ],
        mesh=mesh,
        compiler_params=pltpu.CompilerParams(needs_layout_passes=False),
        scratch_types=[
            pltpu.VMEM((1, CH), jnp.int32),
            pltpu.VMEM((1, CH), jnp.int32),
            pltpu.VMEM((LR, D), jnp.float32),
            pltpu.VMEM((CH, D), jnp.float32),
            pltpu.VMEM((CH, D), jnp.float32),
            pltpu.VMEM((LR, D), jnp.float32),
            pltpu.VMEM((DRR, D), jnp.float32),
            pltpu.VMEM((1, NOFF), jnp.int32),
        ],
    )
    return kern(q, k, v, dst2, src2, off2)


def kernel(x, edge_index, batch, Wq, bq, Wk, bk, Wv, bv, Ws, bs, lin_W, lin_b):
    dsts, srcs = lax.sort((edge_index[1], edge_index[0]), num_keys=1)
    dst2 = jnp.concatenate(
        [dsts, jnp.full((SLACK,), N, jnp.int32)]).reshape(1, EPP)
    src2 = jnp.concatenate(
        [srcs, jnp.zeros((SLACK,), jnp.int32)]).reshape(1, EPP)
    off = jnp.searchsorted(dsts, jnp.arange(NW + 1, dtype=jnp.int32) * LR)
    off2 = jnp.pad(off.astype(jnp.int32), (0, NOFF - (NW + 1)),
                   constant_values=E).reshape(1, NOFF)

    bb = jnp.broadcast_to(batch[:, None], (N, D))
    wcat = jnp.concatenate([Wq, Wk, Wv, Ws], axis=2)
    bcat = jnp.concatenate([bq, bk, bv, bs], axis=1)
    bcat = bcat.reshape(L, 1, 4 * D)

    skip = None
    num = den = None
    for l in range(L):
        if l == 0:
            q, k, v, skip = _run_qkvs_first(x, wcat[0], bcat[0])
        else:
            q, k, v, skip = _run_qkvs_mid(num, den, skip, wcat[l], bcat[l])
        qp = jnp.pad(q, ((0, NPAD - N), (0, 0)))
        num, den_raw = _run_sc_edge(qp, k, v, dst2, src2, off2)
        den = den_raw.reshape(NPAD, 16)
    return _run_final(num, den, skip, bb, lin_W, lin_b.reshape(1, 1))


# speedup vs baseline: 3.3821x; 1.6768x over previous
import functools

import jax
import jax.numpy as jnp
from jax import lax
from jax.experimental import pallas as pl
from jax.experimental.pallas import tpu as pltpu
from jax.experimental.pallas import tpu_sc as plsc

N = 10000
E = 320000
D = 128
G = 16
L = 8

NC = 2
NS = 16
NW = NC * NS
CH = 128
NG = CH // 16
NPAD = 10240
LR = NPAD // NW
DRR = NPAD // NW // 8
SLACK = 128
EPP = E + SLACK
NOFF = 48
RB = 1000
NRB = N // RB

_SELU_SCALE = 1.0507009873554805
_SELU_ALPHA = 1.6732632423543772


def _qkvs_first_body(x_ref, w_ref, b_ref, q_ref, k_ref, v_ref, s_ref):
    h = x_ref[...]
    y = jnp.dot(h, w_ref[...], preferred_element_type=jnp.float32) + b_ref[...]
    q_ref[...] = y[:, 0:D]
    k_ref[...] = y[:, D:2 * D]
    v_ref[...] = y[:, 2 * D:3 * D]
    s_ref[...] = y[:, 3 * D:4 * D]


def _qkvs_mid_body(num_ref, den_ref, skip_ref, w_ref, b_ref,
                   q_ref, k_ref, v_ref, s_ref):
    num = num_ref[...]
    den = den_ref[:, 0:1] + 1e-16
    h = num / den + skip_ref[...]
    h = _SELU_SCALE * jnp.where(h > 0, h, _SELU_ALPHA * (jnp.exp(h) - 1.0))
    y = jnp.dot(h, w_ref[...], preferred_element_type=jnp.float32) + b_ref[...]
    q_ref[...] = y[:, 0:D]
    k_ref[...] = y[:, D:2 * D]
    v_ref[...] = y[:, 2 * D:3 * D]
    s_ref[...] = y[:, 3 * D:4 * D]


def _run_qkvs_first(x, w, b):
    return pl.pallas_call(
        _qkvs_first_body,
        grid=(NRB,),
        in_specs=[
            pl.BlockSpec((RB, D), lambda i: (i, 0)),
            pl.BlockSpec((D, 4 * D), lambda i: (0, 0)),
            pl.BlockSpec((1, 4 * D), lambda i: (0, 0)),
        ],
        out_specs=[pl.BlockSpec((RB, D), lambda i: (i, 0))] * 4,
        out_shape=[jax.ShapeDtypeStruct((N, D), jnp.float32)] * 4,
    )(x, w, b)


def _run_qkvs_mid(num, den, skip, w, b):
    return pl.pallas_call(
        _qkvs_mid_body,
        grid=(NRB,),
        in_specs=[
            pl.BlockSpec((RB, D), lambda i: (i, 0)),
            pl.BlockSpec((RB, 16), lambda i: (i, 0)),
            pl.BlockSpec((RB, D), lambda i: (i, 0)),
            pl.BlockSpec((D, 4 * D), lambda i: (0, 0)),
            pl.BlockSpec((1, 4 * D), lambda i: (0, 0)),
        ],
        out_specs=[pl.BlockSpec((RB, D), lambda i: (i, 0))] * 4,
        out_shape=[jax.ShapeDtypeStruct((N, D), jnp.float32)] * 4,
    )(num, den, skip, w, b)


def _final_body(num_ref, den_ref, skip_ref, bb_ref, lw_ref, lb_ref,
                pooled_ref, out_ref):
    i = pl.program_id(0)
    num = num_ref[...]
    den = den_ref[:, 0:1] + 1e-16
    h = num / den + skip_ref[...]

    @pl.when(i == 0)
    def _():
        pooled_ref[...] = jnp.full((G, D), -jnp.inf, jnp.float32)

    bb = bb_ref[...]
    for g in range(G):
        vals = jnp.where(bb == g, h, -jnp.inf)
        mg = jnp.max(vals, axis=0, keepdims=True)
        pooled_ref[pl.ds(g, 1), :] = jnp.maximum(pooled_ref[pl.ds(g, 1), :], mg)

    @pl.when(i == NRB - 1)
    def _():
        p = pooled_ref[...]
        yv = jnp.dot(p, lw_ref[...], preferred_element_type=jnp.float32)
        yv = yv + lb_ref[...]
        out_ref[...] = 1.0 / (1.0 + jnp.exp(-yv))


def _run_final(num, den, skip, bb, lw, lb):
    _, out = pl.pallas_call(
        _final_body,
        grid=(NRB,),
        in_specs=[
            pl.BlockSpec((RB, D), lambda i: (i, 0)),
            pl.BlockSpec((RB, 16), lambda i: (i, 0)),
            pl.BlockSpec((RB, D), lambda i: (i, 0)),
            pl.BlockSpec((RB, D), lambda i: (i, 0)),
            pl.BlockSpec((D, 1), lambda i: (0, 0)),
            pl.BlockSpec((1, 1), lambda i: (0, 0)),
        ],
        out_specs=[
            pl.BlockSpec((G, D), lambda i: (0, 0)),
            pl.BlockSpec((G, 1), lambda i: (0, 0)),
        ],
        out_shape=[
            jax.ShapeDtypeStruct((G, D), jnp.float32),
            jax.ShapeDtypeStruct((G, 1), jnp.float32),
        ],
    )(num, den, skip, bb, lw, lb)
    return out


def _sc_edge_body(q_hbm, k_hbm, v_hbm, dst_hbm, src_hbm, off_hbm,
                  outnum_hbm, outden_hbm,
                  dstc, srcc, qloc, kbuf, vbuf, accloc, denloc, offb):
    c = lax.axis_index("c")
    s = lax.axis_index("s")
    g = c * NS + s
    base = g * LR
    iota = lax.iota(jnp.int32, 16)
    z16 = jnp.zeros((16,), jnp.float32)
    lane0 = iota == 0

    pltpu.sync_copy(off_hbm, offb)

    def _scalar_at(pos):
        w = offb[0, pl.ds((pos // 16) * 16, 16)]
        spl = w.at[jnp.broadcast_to(pos % 16, (16,)).astype(jnp.int32)].get(
            mode=lax.GatherScatterMode.PROMISE_IN_BOUNDS)
        return spl[0]

    off0 = _scalar_at(g)
    off1 = _scalar_at(g + 1)
    off0a = lax.bitwise_and(off0, jnp.int32(~127))
    nch = (off1 - off0a + (CH - 1)) // CH

    def zrow(i, carry):
        for t in range(D // 16):
            accloc[i, pl.ds(t * 16, 16)] = z16
        return carry

    lax.fori_loop(0, LR, zrow, 0)

    def zden(i, carry):
        for t in range(D // 16):
            denloc[i, pl.ds(t * 16, 16)] = z16
        return carry

    lax.fori_loop(0, DRR, zden, 0)

    pltpu.sync_copy(q_hbm.at[pl.ds(base, LR)], qloc)

    inv = jnp.float32(1.0 / (D ** 0.5))

    def chunk(j, carry):
        st = pl.multiple_of(off0a + j * CH, CH)
        pltpu.sync_copy(dst_hbm.at[:, pl.ds(st, CH)], dstc)
        pltpu.sync_copy(src_hbm.at[:, pl.ds(st, CH)], srcc)
        idx_s = srcc.at[0]
        pltpu.sync_copy(k_hbm.at[idx_s], kbuf)
        pltpu.sync_copy(v_hbm.at[idx_s], vbuf)
        lane15 = jnp.full((16,), 15, jnp.int32)

        def group(gi, carry2):
            gb = gi * 16
            dstv = dstc[0, pl.ds(gb, 16)]
            ldv = jnp.minimum(jnp.maximum(dstv - base, 0), LR - 1)
            for e in range(16):
                ld = ldv[e]
                r = gb + e
                m = [qloc[ld, pl.ds(t * 16, 16)] * kbuf[r, pl.ds(t * 16, 16)]
                     for t in range(D // 16)]
                m = [m[0] + m[1], m[2] + m[3], m[4] + m[5], m[6] + m[7]]
                acc = (m[0] + m[1]) + (m[2] + m[3])
                spl = plsc.cumsum(acc).at[lane15].get(
                    mode=lax.GatherScatterMode.PROMISE_IN_BOUNDS)
                eix = st + r
                okv = jnp.broadcast_to(
                    jnp.logical_and(eix >= off0, eix < off1), (16,))
                exs = jnp.where(okv, jnp.exp(spl * inv), z16)
                for t in range(D // 16):
                    plsc.addupdate(accloc.at[ld, pl.ds(t * 16, 16)],
                                   vbuf[r, pl.ds(t * 16, 16)] * exs)
                ldr = lax.shift_right_logical(ld, 3)
                ldc = pl.multiple_of(lax.bitwise_and(ld, 7) * 16, 16)
                plsc.addupdate(denloc.at[ldr, pl.ds(ldc, 16)],
                               jnp.where(lane0, exs, jnp.float32(0.0)))
            return carry2

        lax.fori_loop(0, NG, group, 0)
        return carry

    lax.fori_loop(0, nch, chunk, 0)
    pltpu.sync_copy(accloc, outnum_hbm.at[pl.ds(base, LR)])
    pltpu.sync_copy(denloc, outden_hbm.at[pl.ds(g * DRR, DRR)])


def _run_sc_edge(q, k, v, dst2, src2, off2):
    mesh = plsc.VectorSubcoreMesh(core_axis_name="c", subcore_axis_name="s",
                                  num_cores=NC, num_subcores=NS)
    kern = pl.kernel(
        _sc_edge_body,
        out_type=[
            jax.ShapeDtypeStruct((NPAD, D), jnp.float32),
            jax.ShapeDtypeStruct((NPAD // 8, D), jnp.float32),
        ],
        mesh=mesh,
        compiler_params=pltpu.CompilerParams(needs_layout_passes=False),
        scratch_types=[
            pltpu.VMEM((1, CH), jnp.int32),
            pltpu.VMEM((1, CH), jnp.int32),
            pltpu.VMEM((LR, D), jnp.float32),
            pltpu.VMEM((CH, D), jnp.float32),
            pltpu.VMEM((CH, D), jnp.float32),
            pltpu.VMEM((LR, D), jnp.float32),
            pltpu.VMEM((DRR, D), jnp.float32),
            pltpu.VMEM((1, NOFF), jnp.int32),
        ],
    )
    return kern(q, k, v, dst2, src2, off2)


def kernel(x, edge_index, batch, Wq, bq, Wk, bk, Wv, bv, Ws, bs, lin_W, lin_b):
    dsts, srcs = lax.sort((edge_index[1], edge_index[0]), num_keys=1)
    dst2 = jnp.concatenate(
        [dsts, jnp.full((SLACK,), N, jnp.int32)]).reshape(1, EPP)
    src2 = jnp.concatenate(
        [srcs, jnp.zeros((SLACK,), jnp.int32)]).reshape(1, EPP)
    off = jnp.searchsorted(dsts, jnp.arange(NW + 1, dtype=jnp.int32) * LR)
    off2 = jnp.pad(off.astype(jnp.int32), (0, NOFF - (NW + 1)),
                   constant_values=E).reshape(1, NOFF)

    bb = jnp.broadcast_to(batch[:, None], (N, D))
    wcat = jnp.concatenate([Wq, Wk, Wv, Ws], axis=2)
    bcat = jnp.concatenate([bq, bk, bv, bs], axis=1)
    bcat = bcat.reshape(L, 1, 4 * D)

    skip = None
    num = den = None
    for l in range(L):
        if l == 0:
            q, k, v, skip = _run_qkvs_first(x, wcat[0], bcat[0])
        else:
            q, k, v, skip = _run_qkvs_mid(num, den, skip, wcat[l], bcat[l])
        qp = jnp.pad(q, ((0, NPAD - N), (0, 0)))
        num, den_raw = _run_sc_edge(qp, k, v, dst2, src2, off2)
        den = den_raw.reshape(NPAD, 16)
    return _run_final(num, den, skip, bb, lin_W, lin_b.reshape(1, 1))


# speedup vs baseline: 3.7057x; 1.0957x over previous
import functools

import jax
import jax.numpy as jnp
from jax import lax
from jax.experimental import pallas as pl
from jax.experimental.pallas import tpu as pltpu
from jax.experimental.pallas import tpu_sc as plsc

N = 10000
E = 320000
D = 128
G = 16
L = 8

NC = 2
NS = 16
NW = NC * NS
CH = 128
NG = CH // 16
NPAD = 10240
LR = NPAD // NW
DRR = NPAD // NW // 8
SLACK = 128
EPP = E + SLACK
NOFF = 48
RB = 1000
NRB = N // RB

_SELU_SCALE = 1.0507009873554805
_SELU_ALPHA = 1.6732632423543772


def _qkvs_first_body(x_ref, w_ref, b_ref, q_ref, k_ref, v_ref, s_ref):
    h = x_ref[...]
    y = jnp.dot(h, w_ref[...], preferred_element_type=jnp.float32) + b_ref[...]
    q_ref[...] = y[:, 0:D]
    k_ref[...] = y[:, D:2 * D]
    v_ref[...] = y[:, 2 * D:3 * D]
    s_ref[...] = y[:, 3 * D:4 * D]


def _qkvs_mid_body(num_ref, den_ref, skip_ref, w_ref, b_ref,
                   q_ref, k_ref, v_ref, s_ref):
    num = num_ref[...]
    den = den_ref[:, 0:1] + 1e-16
    h = num / den + skip_ref[...]
    h = _SELU_SCALE * jnp.where(h > 0, h, _SELU_ALPHA * (jnp.exp(h) - 1.0))
    y = jnp.dot(h, w_ref[...], preferred_element_type=jnp.float32) + b_ref[...]
    q_ref[...] = y[:, 0:D]
    k_ref[...] = y[:, D:2 * D]
    v_ref[...] = y[:, 2 * D:3 * D]
    s_ref[...] = y[:, 3 * D:4 * D]


def _run_qkvs_first(x, w, b):
    return pl.pallas_call(
        _qkvs_first_body,
        grid=(NRB,),
        in_specs=[
            pl.BlockSpec((RB, D), lambda i: (i, 0)),
            pl.BlockSpec((D, 4 * D), lambda i: (0, 0)),
            pl.BlockSpec((1, 4 * D), lambda i: (0, 0)),
        ],
        out_specs=[pl.BlockSpec((RB, D), lambda i: (i, 0))] * 4,
        out_shape=[jax.ShapeDtypeStruct((N, D), jnp.float32)] * 4,
    )(x, w, b)


def _run_qkvs_mid(num, den, skip, w, b):
    return pl.pallas_call(
        _qkvs_mid_body,
        grid=(NRB,),
        in_specs=[
            pl.BlockSpec((RB, D), lambda i: (i, 0)),
            pl.BlockSpec((RB, 16), lambda i: (i, 0)),
            pl.BlockSpec((RB, D), lambda i: (i, 0)),
            pl.BlockSpec((D, 4 * D), lambda i: (0, 0)),
            pl.BlockSpec((1, 4 * D), lambda i: (0, 0)),
        ],
        out_specs=[pl.BlockSpec((RB, D), lambda i: (i, 0))] * 4,
        out_shape=[jax.ShapeDtypeStruct((N, D), jnp.float32)] * 4,
    )(num, den, skip, w, b)


def _final_body(num_ref, den_ref, skip_ref, bb_ref, lw_ref, lb_ref,
                pooled_ref, out_ref):
    i = pl.program_id(0)
    num = num_ref[...]
    den = den_ref[:, 0:1] + 1e-16
    h = num / den + skip_ref[...]

    @pl.when(i == 0)
    def _():
        pooled_ref[...] = jnp.full((G, D), -jnp.inf, jnp.float32)

    bb = bb_ref[...]
    for g in range(G):
        vals = jnp.where(bb == g, h, -jnp.inf)
        mg = jnp.max(vals, axis=0, keepdims=True)
        pooled_ref[pl.ds(g, 1), :] = jnp.maximum(pooled_ref[pl.ds(g, 1), :], mg)

    @pl.when(i == NRB - 1)
    def _():
        p = pooled_ref[...]
        yv = jnp.dot(p, lw_ref[...], preferred_element_type=jnp.float32)
        yv = yv + lb_ref[...]
        out_ref[...] = 1.0 / (1.0 + jnp.exp(-yv))


def _run_final(num, den, skip, bb, lw, lb):
    _, out = pl.pallas_call(
        _final_body,
        grid=(NRB,),
        in_specs=[
            pl.BlockSpec((RB, D), lambda i: (i, 0)),
            pl.BlockSpec((RB, 16), lambda i: (i, 0)),
            pl.BlockSpec((RB, D), lambda i: (i, 0)),
            pl.BlockSpec((RB, D), lambda i: (i, 0)),
            pl.BlockSpec((D, 1), lambda i: (0, 0)),
            pl.BlockSpec((1, 1), lambda i: (0, 0)),
        ],
        out_specs=[
            pl.BlockSpec((G, D), lambda i: (0, 0)),
            pl.BlockSpec((G, 1), lambda i: (0, 0)),
        ],
        out_shape=[
            jax.ShapeDtypeStruct((G, D), jnp.float32),
            jax.ShapeDtypeStruct((G, 1), jnp.float32),
        ],
    )(num, den, skip, bb, lw, lb)
    return out


def _sc_edge_body(q_hbm, k_hbm, v_hbm, dst_hbm, src_hbm, off_hbm,
                  outnum_hbm, outden_hbm,
                  dstc, srcc, qloc, kbuf, vbuf, accloc, denloc, offb,
                  semd, sems, semk, semv):
    c = lax.axis_index("c")
    s = lax.axis_index("s")
    g = c * NS + s
    base = g * LR
    iota = lax.iota(jnp.int32, 16)
    z16 = jnp.zeros((16,), jnp.float32)
    lane0 = iota == 0

    pltpu.sync_copy(off_hbm, offb)

    def _scalar_at(pos):
        w = offb[0, pl.ds((pos // 16) * 16, 16)]
        spl = w.at[jnp.broadcast_to(pos % 16, (16,)).astype(jnp.int32)].get(
            mode=lax.GatherScatterMode.PROMISE_IN_BOUNDS)
        return spl[0]

    off0 = _scalar_at(g)
    off1 = _scalar_at(g + 1)
    off0a = lax.bitwise_and(off0, jnp.int32(~127))
    nch = (off1 - off0a + (CH - 1)) // CH

    def zrow(i, carry):
        for t in range(D // 16):
            accloc[i, pl.ds(t * 16, 16)] = z16
        return carry

    lax.fori_loop(0, LR, zrow, 0)

    def zden(i, carry):
        for t in range(D // 16):
            denloc[i, pl.ds(t * 16, 16)] = z16
        return carry

    lax.fori_loop(0, DRR, zden, 0)

    pltpu.sync_copy(q_hbm.at[pl.ds(base, LR)], qloc)

    inv = jnp.float32(1.0 / (D ** 0.5))

    def chunk(j, carry):
        st = pl.multiple_of(off0a + j * CH, CH)
        cpd = pltpu.async_copy(dst_hbm.at[:, pl.ds(st, CH)], dstc, semd)
        cps = pltpu.async_copy(src_hbm.at[:, pl.ds(st, CH)], srcc, sems)
        cpd.wait()
        cps.wait()
        idx_s = srcc.at[0]
        cpk = pltpu.async_copy(k_hbm.at[idx_s], kbuf, semk)
        cpv = pltpu.async_copy(v_hbm.at[idx_s], vbuf, semv)
        cpk.wait()
        cpv.wait()
        lane15 = jnp.full((16,), 15, jnp.int32)

        def group(gi, carry2):
            gb = gi * 16
            dstv = dstc[0, pl.ds(gb, 16)]
            ldv = jnp.minimum(jnp.maximum(dstv - base, 0), LR - 1)
            for e in range(16):
                ld = ldv[e]
                r = gb + e
                m = [qloc[ld, pl.ds(t * 16, 16)] * kbuf[r, pl.ds(t * 16, 16)]
                     for t in range(D // 16)]
                m = [m[0] + m[1], m[2] + m[3], m[4] + m[5], m[6] + m[7]]
                acc = (m[0] + m[1]) + (m[2] + m[3])
                spl = plsc.cumsum(acc).at[lane15].get(
                    mode=lax.GatherScatterMode.PROMISE_IN_BOUNDS)
                eix = st + r
                okv = jnp.broadcast_to(
                    jnp.logical_and(eix >= off0, eix < off1), (16,))
                exs = jnp.where(okv, jnp.exp(spl * inv), z16)
                for t in range(D // 16):
                    plsc.addupdate(accloc.at[ld, pl.ds(t * 16, 16)],
                                   vbuf[r, pl.ds(t * 16, 16)] * exs)
                ldr = lax.shift_right_logical(ld, 3)
                ldc = pl.multiple_of(lax.bitwise_and(ld, 7) * 16, 16)
                plsc.addupdate(denloc.at[ldr, pl.ds(ldc, 16)],
                               jnp.where(lane0, exs, jnp.float32(0.0)))
            return carry2

        lax.fori_loop(0, NG, group, 0)
        return carry

    lax.fori_loop(0, nch, chunk, 0)
    pltpu.sync_copy(accloc, outnum_hbm.at[pl.ds(base, LR)])
    pltpu.sync_copy(denloc, outden_hbm.at[pl.ds(g * DRR, DRR)])


def _run_sc_edge(q, k, v, dst2, src2, off2):
    mesh = plsc.VectorSubcoreMesh(core_axis_name="c", subcore_axis_name="s",
                                  num_cores=NC, num_subcores=NS)
    kern = pl.kernel(
        _sc_edge_body,
        out_type=[
            jax.ShapeDtypeStruct((NPAD, D), jnp.float32),
            jax.ShapeDtypeStruct((NPAD // 8, D), jnp.float32),
        ],
        mesh=mesh,
        compiler_params=pltpu.CompilerParams(needs_layout_passes=False),
        scratch_types=[
            pltpu.VMEM((1, CH), jnp.int32),
            pltpu.VMEM((1, CH), jnp.int32),
            pltpu.VMEM((LR, D), jnp.float32),
            pltpu.VMEM((CH, D), jnp.float32),
            pltpu.VMEM((CH, D), jnp.float32),
            pltpu.VMEM((LR, D), jnp.float32),
            pltpu.VMEM((DRR, D), jnp.float32),
            pltpu.VMEM((1, NOFF), jnp.int32),
            pltpu.SemaphoreType.DMA,
            pltpu.SemaphoreType.DMA,
            pltpu.SemaphoreType.DMA,
            pltpu.SemaphoreType.DMA,
        ],
    )
    return kern(q, k, v, dst2, src2, off2)


def kernel(x, edge_index, batch, Wq, bq, Wk, bk, Wv, bv, Ws, bs, lin_W, lin_b):
    dsts, srcs = lax.sort((edge_index[1], edge_index[0]), num_keys=1)
    dst2 = jnp.concatenate(
        [dsts, jnp.full((SLACK,), N, jnp.int32)]).reshape(1, EPP)
    src2 = jnp.concatenate(
        [srcs, jnp.zeros((SLACK,), jnp.int32)]).reshape(1, EPP)
    off = jnp.searchsorted(dsts, jnp.arange(NW + 1, dtype=jnp.int32) * LR)
    off2 = jnp.pad(off.astype(jnp.int32), (0, NOFF - (NW + 1)),
                   constant_values=E).reshape(1, NOFF)

    bb = jnp.broadcast_to(batch[:, None], (N, D))
    wcat = jnp.concatenate([Wq, Wk, Wv, Ws], axis=2)
    bcat = jnp.concatenate([bq, bk, bv, bs], axis=1)
    bcat = bcat.reshape(L, 1, 4 * D)

    skip = None
    num = den = None
    for l in range(L):
        if l == 0:
            q, k, v, skip = _run_qkvs_first(x, wcat[0], bcat[0])
        else:
            q, k, v, skip = _run_qkvs_mid(num, den, skip, wcat[l], bcat[l])
        qp = jnp.pad(q, ((0, NPAD - N), (0, 0)))
        num, den_raw = _run_sc_edge(qp, k, v, dst2, src2, off2)
        den = den_raw.reshape(NPAD, 16)
    return _run_final(num, den, skip, bb, lin_W, lin_b.reshape(1, 1))
